# baseline (speedup 1.0000x reference)
import jax
import jax.numpy as jnp
from jax.experimental import pallas as pl
from jax.experimental.pallas import tpu as pltpu

_NUM_NODES = 100000
_D_MOD = 32
_D_OUT = 128
_BLOCK = 4000


def _fused_gcn_body(a_ref, v_ref, x1_ref, x2_ref, w1_ref, b1_ref, w2_ref,
                    b2_ref, out_ref):
    f32 = jnp.float32
    xw = (
        jnp.dot(a_ref[...], w1_ref[0:32, :], preferred_element_type=f32)
        + jnp.dot(v_ref[...], w1_ref[32:64, :], preferred_element_type=f32)
        + jnp.dot(x1_ref[...], w1_ref[64:96, :], preferred_element_type=f32)
        + jnp.dot(x2_ref[...], w1_ref[96:128, :], preferred_element_type=f32)
    )
    b1 = b1_ref[...]
    h = xw + b1
    m = 0.25 * jnp.sum(xw[0:4, :], axis=0, keepdims=True) + b1
    row = jax.lax.broadcasted_iota(jnp.int32, h.shape, 0)
    is_hub = jnp.logical_and(pl.program_id(0) == 0, row < 4)
    h = jnp.maximum(jnp.where(is_hub, m, h), 0.0)
    out_ref[...] = (
        jnp.dot(h, w2_ref[...], preferred_element_type=f32) + b2_ref[...]
    )


def kernel(audio, video, x1, x2, W_gcn1, b_gcn1, W_gcn2, b_gcn2):
    w1 = W_gcn1.T
    w2 = W_gcn2.T
    b1 = b_gcn1.reshape(1, _D_OUT)
    b2 = b_gcn2.reshape(1, _D_OUT)
    grid = (_NUM_NODES // _BLOCK,)
    mod_spec = pl.BlockSpec((_BLOCK, _D_MOD), lambda i: (i, 0))
    full_spec = pl.BlockSpec((4 * _D_MOD, _D_OUT), lambda i: (0, 0))
    w2_spec = pl.BlockSpec((_D_OUT, _D_OUT), lambda i: (0, 0))
    b_spec = pl.BlockSpec((1, _D_OUT), lambda i: (0, 0))
    return pl.pallas_call(
        _fused_gcn_body,
        grid=grid,
        in_specs=[mod_spec, mod_spec, mod_spec, mod_spec,
                  full_spec, b_spec, w2_spec, b_spec],
        out_specs=pl.BlockSpec((_BLOCK, _D_OUT), lambda i: (i, 0)),
        out_shape=jax.ShapeDtypeStruct((_NUM_NODES, _D_OUT), jnp.float32),
        compiler_params=pltpu.CompilerParams(
            dimension_semantics=("arbitrary",),
        ),
    )(audio, video, x1, x2, w1, b1, w2, b2)


# speedup vs baseline: 1.0016x; 1.0016x over previous
import jax
import jax.numpy as jnp
from jax.experimental import pallas as pl
from jax.experimental.pallas import tpu as pltpu

_NUM_NODES = 100000
_D_MOD = 32
_D_OUT = 128
_BLOCK = 4000


def _fused_gcn_body(a_ref, v_ref, x1_ref, x2_ref, w1_ref, b1_ref, w2_ref,
                    b2_ref, out_ref):
    f32 = jnp.float32
    xw = (
        jnp.dot(a_ref[...], w1_ref[0:32, :], preferred_element_type=f32)
        + jnp.dot(v_ref[...], w1_ref[32:64, :], preferred_element_type=f32)
        + jnp.dot(x1_ref[...], w1_ref[64:96, :], preferred_element_type=f32)
        + jnp.dot(x2_ref[...], w1_ref[96:128, :], preferred_element_type=f32)
    )
    b1 = b1_ref[...]
    h = xw + b1
    m = 0.25 * jnp.sum(xw[0:4, :], axis=0, keepdims=True) + b1
    row = jax.lax.broadcasted_iota(jnp.int32, h.shape, 0)
    is_hub = jnp.logical_and(pl.program_id(0) == 0, row < 4)
    h = jnp.maximum(jnp.where(is_hub, m, h), 0.0)
    out_ref[...] = (
        jnp.dot(h, w2_ref[...], preferred_element_type=f32) + b2_ref[...]
    )


def kernel(audio, video, x1, x2, W_gcn1, b_gcn1, W_gcn2, b_gcn2):
    w1 = W_gcn1.T
    w2 = W_gcn2.T
    b1 = b_gcn1.reshape(1, _D_OUT)
    b2 = b_gcn2.reshape(1, _D_OUT)
    grid = (_NUM_NODES // _BLOCK,)
    mod_spec = pl.BlockSpec((_BLOCK, _D_MOD), lambda i: (i, 0))
    full_spec = pl.BlockSpec((4 * _D_MOD, _D_OUT), lambda i: (0, 0))
    w2_spec = pl.BlockSpec((_D_OUT, _D_OUT), lambda i: (0, 0))
    b_spec = pl.BlockSpec((1, _D_OUT), lambda i: (0, 0))
    return pl.pallas_call(
        _fused_gcn_body,
        grid=grid,
        in_specs=[mod_spec, mod_spec, mod_spec, mod_spec,
                  full_spec, b_spec, w2_spec, b_spec],
        out_specs=pl.BlockSpec((_BLOCK, _D_OUT), lambda i: (i, 0)),
        out_shape=jax.ShapeDtypeStruct((_NUM_NODES, _D_OUT), jnp.float32),
        compiler_params=pltpu.CompilerParams(
            dimension_semantics=("parallel",),
        ),
    )(audio, video, x1, x2, w1, b1, w2, b2)
